# baseline (speedup 1.0000x reference)
import functools

import jax
import jax.numpy as jnp
from jax import lax
from jax.experimental import pallas as pl
from jax.experimental.pallas import tpu as pltpu
from jax.experimental.pallas import tpu_sc as plsc

N = 10000
HID = 128
HH = HID // 2
NT = 16
ROWS = 10240
RPT = ROWS // NT

_MESH = plsc.VectorSubcoreMesh(core_axis_name="c", subcore_axis_name="s")


def _sc_embed(emb2, hidx):
    vocab = emb2.shape[0] // 2

    @functools.partial(
        pl.kernel,
        out_type=jax.ShapeDtypeStruct((2 * ROWS, HH), jnp.float32),
        mesh=_MESH,
        compiler_params=pltpu.CompilerParams(use_tc_tiling_on_sc=False),
        scratch_types=[
            pltpu.VMEM((5, 128), jnp.int32),
            pltpu.VMEM((128, HH), jnp.float32),
        ],
    )
    def k(emb_hbm, hidx_hbm, x_hbm, idx_v, rows_v):
        cid = lax.axis_index("c")
        sid = lax.axis_index("s")
        pltpu.sync_copy(hidx_hbm.at[sid], idx_v)
        off = (cid * vocab).astype(jnp.int32)

        @pl.loop(0, 5)
        def _off(c):
            for j in range(8):
                s = pl.ds(j * 16, 16)
                idx_v[c, s] = idx_v[c, s] + off

        for c in range(5):
            pltpu.sync_copy(emb_hbm.at[idx_v.at[c]], rows_v)
            pltpu.sync_copy(
                rows_v,
                x_hbm.at[pl.ds(cid * ROWS + sid * RPT + c * 128, 128)])

    return k(emb2, hidx)


NB = 64


def _sc_agg(x2, src, dst, nchunk, page_rows):
    nblk = nchunk // NB

    @functools.partial(
        pl.kernel,
        out_type=jax.ShapeDtypeStruct((2 * ROWS, HH), jnp.float32),
        mesh=_MESH,
        compiler_params=pltpu.CompilerParams(use_tc_tiling_on_sc=False),
        scratch_types=[
            pltpu.VMEM((NB, 128), jnp.int32),
            pltpu.VMEM((NB, 128), jnp.int32),
            [pltpu.VMEM((128, HH), jnp.float32)] * 8,
            pltpu.VMEM((64, HH), jnp.float32),
            pltpu.VMEM_SHARED((ROWS, HH), jnp.float32),
            [pltpu.SemaphoreType.DMA] * 8,
            [pltpu.SemaphoreType.DMA] * 8,
        ],
    )
    def k(x_hbm, src_hbm, dst_hbm, agg_hbm, srcs_v, dsts_v, rows,
          zbuf, agg_sh, gsem, ssem):
        cid = lax.axis_index("c")
        sid = lax.axis_index("s")

        @pl.loop(0, 64)
        def _zero(r):
            for j in range(HH // 16):
                zbuf[r, pl.ds(j * 16, 16)] = jnp.zeros((16,), jnp.float32)

        @pl.loop(0, RPT // 64)
        def _init(t):
            pltpu.sync_copy(zbuf, agg_sh.at[pl.ds(sid * RPT + t * 64, 64)])

        plsc.subcore_barrier()
        off = (cid * page_rows).astype(jnp.int32)

        @pl.loop(0, nblk)
        def _blk(b):
            base = sid * nchunk + b * NB
            pltpu.sync_copy(src_hbm.at[pl.ds(base, NB)], srcs_v)
            pltpu.sync_copy(dst_hbm.at[pl.ds(base, NB)], dsts_v)

            @pl.loop(0, NB)
            def _off(c):
                for j in range(8):
                    s = pl.ds(j * 16, 16)
                    srcs_v[c, s] = srcs_v[c, s] + off

            def fire_g(c, q):
                pltpu.async_copy(x_hbm.at[srcs_v.at[c]], rows[q], gsem[q])

            def drain_g(c, q):
                pltpu.make_async_copy(x_hbm.at[srcs_v.at[c]], rows[q],
                                      gsem[q]).wait()

            def fire_s(c, q):
                pltpu.async_copy(rows[q], agg_sh.at[dsts_v.at[c]], ssem[q],
                                 add=True)

            def drain_s(c, q):
                pltpu.make_async_copy(rows[q], agg_sh.at[dsts_v.at[c]],
                                      ssem[q]).wait()

            for q0 in range(4):
                fire_g(jnp.int32(q0), q0)

            @pl.loop(0, NB // 8)
            def _edges(c8):
                for kq in range(8):
                    c = 8 * c8 + kq
                    drain_g(c, kq)
                    if kq < 4:
                        @pl.when(c >= 4)
                        def _():
                            drain_s(c - 4, kq + 4)

                        fire_g(c + 4, kq + 4)
                    else:
                        drain_s(c - 4, kq - 4)

                        @pl.when(c + 4 < NB)
                        def _():
                            fire_g(c + 4, kq - 4)

                    fire_s(c, kq)

            for q0 in range(4):
                drain_s(jnp.int32(NB - 4 + q0), 4 + q0)

        plsc.subcore_barrier()
        pltpu.sync_copy(agg_sh.at[pl.ds(sid * RPT, RPT)],
                        agg_hbm.at[pl.ds(cid * ROWS + sid * RPT, RPT)])

    return k(x2, src, dst)


def _dot_bf(x, w):
    return jnp.dot(x.astype(jnp.bfloat16), w.astype(jnp.bfloat16).T,
                   preferred_element_type=jnp.float32)


def _bn_relu(hh, g, b, relu=True):
    m = jnp.mean(hh, axis=0, keepdims=True)
    v = jnp.mean((hh - m) ** 2, axis=0, keepdims=True)
    out = (hh - m) * lax.rsqrt(v + 1e-5) * g + b
    return jnp.maximum(out, 0.0) if relu else out


def _split_cols(ref, rows):
    return jnp.concatenate([ref[0:N, :], ref[rows:rows + N, :]], axis=1)


def _tc_layer1(x2, agg, w0, w1, g1, b1, g2, b2):
    def body(x_ref, agg_ref, w0_ref, w1_ref, g1_ref, b1_ref, g2_ref, b2_ref,
             out_ref):
        xv = _split_cols(x_ref, ROWS)
        rst = 2.0 * xv + _split_cols(agg_ref, ROWS)
        hh = _dot_bf(rst, w0_ref[...])
        hh = _bn_relu(hh, g1_ref[...], b1_ref[...])
        hh = _dot_bf(hh, w1_ref[...])
        hh = _bn_relu(hh, g2_ref[...], b2_ref[...], relu=False)
        x1 = jnp.maximum(hh, 0.0)
        out_ref[0:N, :] = x1[:, :HH]
        out_ref[N:2 * N, :] = x1[:, HH:]

    return pl.pallas_call(
        body, out_shape=jax.ShapeDtypeStruct((2 * N, HH), jnp.float32),
    )(x2, agg, w0, w1, g1, b1, g2, b2)


def _tc_layer2(x2, x12, agg, w0, w1, g1, b1, g2, b2, wu, wv, wb):
    def body(x_ref, x1_ref, agg_ref, w0_ref, w1_ref, g1_ref, b1_ref, g2_ref,
             b2_ref, wu_ref, wv_ref, wb_ref, a_ref, b_ref):
        x1v = _split_cols(x1_ref, N)
        rst = 2.0 * x1v + _split_cols(agg_ref, ROWS)
        hh = _dot_bf(rst, w0_ref[...])
        hh = _bn_relu(hh, g1_ref[...], b1_ref[...])
        hh = _dot_bf(hh, w1_ref[...])
        hh = _bn_relu(hh, g2_ref[...], b2_ref[...], relu=False)
        x2v = jnp.maximum(hh, 0.0)
        xsum = _split_cols(x_ref, ROWS) + x1v + x2v
        a_ref[...] = _dot_bf(xsum, wu_ref[...]) + wb_ref[...]
        b_ref[...] = _dot_bf(xsum, wv_ref[...])

    return pl.pallas_call(
        body,
        out_shape=[jax.ShapeDtypeStruct((N, HID), jnp.float32),
                   jax.ShapeDtypeStruct((N, HID), jnp.float32)],
    )(x2, x12, agg, w0, w1, g1, b1, g2, b2, wu, wv, wb)


def _sc_score(a, b, u2, v2, nchunk):
    e_out = nchunk * 128
    qq, rr = divmod(nchunk, 32)
    niter = qq + (1 if rr else 0)
    npair = (niter + 2) // 2

    @functools.partial(
        pl.kernel,
        out_type=jax.ShapeDtypeStruct((e_out, HID), jnp.float32),
        mesh=_MESH,
        compiler_params=pltpu.CompilerParams(use_tc_tiling_on_sc=False),
        scratch_types=[
            [pltpu.VMEM((128,), jnp.int32)] * 2,
            [pltpu.VMEM((128,), jnp.int32)] * 2,
            [pltpu.VMEM((128, HID), jnp.float32)] * 2,
            [pltpu.VMEM((128, HID), jnp.float32)] * 2,
            [pltpu.SemaphoreType.DMA] * 2,
            [pltpu.SemaphoreType.DMA] * 2,
            [pltpu.SemaphoreType.DMA] * 2,
        ],
    )
    def k(a_hbm, b_hbm, u_hbm, v_hbm, out_hbm, uidx, vidx, bufa, bufb,
          isem, gsem, osem):
        w = lax.axis_index("s") * 2 + lax.axis_index("c")
        cnt = jnp.where(w < rr, qq + 1, qq).astype(jnp.int32)
        base = (w * qq + jnp.minimum(w, rr)).astype(jnp.int32)

        def fire_idx(i, p):
            @pl.when(i < cnt)
            def _():
                g = base + i
                pltpu.async_copy(u_hbm.at[g], uidx[p], isem[p])
                pltpu.async_copy(v_hbm.at[g], vidx[p], isem[p])

        def drain_idx(i, p):
            @pl.when(i < cnt)
            def _():
                g = base + i
                pltpu.make_async_copy(u_hbm.at[g], uidx[p], isem[p]).wait()
                pltpu.make_async_copy(v_hbm.at[g], vidx[p], isem[p]).wait()

        def fire_gath(i, p):
            @pl.when(i < cnt)
            def _():
                pltpu.async_copy(a_hbm.at[uidx[p]], bufa[p], gsem[p])
                pltpu.async_copy(b_hbm.at[vidx[p]], bufb[p], gsem[p])

        def drain_gath(i, p):
            @pl.when(i < cnt)
            def _():
                pltpu.make_async_copy(a_hbm.at[uidx[p]], bufa[p],
                                      gsem[p]).wait()
                pltpu.make_async_copy(b_hbm.at[vidx[p]], bufb[p],
                                      gsem[p]).wait()

        def fire_out(i, p):
            @pl.when(i < cnt)
            def _():
                g = base + i
                pltpu.async_copy(bufa[p], out_hbm.at[pl.ds(g * 128, 128)],
                                 osem[p])

        def drain_out(i, p):
            @pl.when((i >= 0) & (i < cnt))
            def _():
                g = base + i
                pltpu.make_async_copy(bufa[p],
                                      out_hbm.at[pl.ds(g * 128, 128)],
                                      osem[p]).wait()

        def step(i, p):
            drain_out(i - 1, 1 - p)
            drain_idx(i + 1, 1 - p)
            fire_gath(i + 1, 1 - p)
            drain_gath(i, p)
            fire_idx(i + 2, p)

            @pl.when(i < cnt)
            def _():
                @pl.loop(0, 32)
                def _add(r4):
                    for rr2 in range(4):
                        r = 4 * r4 + rr2
                        for j in range(8):
                            s = pl.ds(j * 16, 16)
                            bufa[p][r, s] = bufa[p][r, s] + bufb[p][r, s]

            fire_out(i, p)

        fire_idx(jnp.int32(0), 0)
        fire_idx(jnp.int32(1), 1)
        drain_idx(jnp.int32(0), 0)
        fire_gath(jnp.int32(0), 0)

        @pl.loop(0, npair)
        def _pair(c2):
            i = 2 * c2
            step(i, 0)
            step(i + 1, 1)

        last = jnp.int32(2 * npair - 1)
        drain_out(last, 1)

    return k(a, b, u2, v2)


def kernel(params, h, edge_index):
    u = edge_index[0]
    v = edge_index[1]
    E = u.shape[0]
    emb = params["emb"]
    vocab = emb.shape[0]

    emb2 = jnp.concatenate([emb[:, :HH], emb[:, HH:]], axis=0)
    h_pad = jnp.concatenate(
        [h, jnp.zeros((ROWS - N,), jnp.int32)]).reshape(NT, 5, 128)
    x2 = _sc_embed(emb2, h_pad)

    ee = 2 * E
    grain = NT * NB * 128
    nchunk = NB * ((ee + grain - 1) // grain)
    pad = NT * nchunk * 128 - ee
    pad_src = jnp.arange(pad, dtype=jnp.int32) % N
    pad_dst = N + (jnp.arange(pad, dtype=jnp.int32) % (ROWS - N))
    src = jnp.concatenate([u, v, pad_src]).reshape(NT * nchunk, 128)
    dst = jnp.concatenate([v, u, pad_dst]).reshape(NT * nchunk, 128)

    agg1 = _sc_agg(x2, src, dst, nchunk, ROWS)
    x12 = _tc_layer1(x2, agg1, params["w0_0"], params["w1_0"],
                     params["bng_0"].reshape(1, HID),
                     params["bnb_0"].reshape(1, HID),
                     params["obng_0"].reshape(1, HID),
                     params["obnb_0"].reshape(1, HID))
    agg2 = _sc_agg(x12, src, dst, nchunk, N)
    wu = params["W_w"][:, :HID]
    wv = params["W_w"][:, HID:]
    a_t, b_t = _tc_layer2(x2, x12, agg2, params["w0_1"], params["w1_1"],
                          params["bng_1"].reshape(1, HID),
                          params["bnb_1"].reshape(1, HID),
                          params["obng_1"].reshape(1, HID),
                          params["obnb_1"].reshape(1, HID),
                          wu, wv, params["W_b"].reshape(1, HID))

    nc_e = E // 128
    u2 = u.reshape(nc_e, 128)
    v2 = v.reshape(nc_e, 128)
    return _sc_score(a_t, b_t, u2, v2, nc_e)


# speedup vs baseline: 1.0053x; 1.0053x over previous
import functools

import jax
import jax.numpy as jnp
from jax import lax
from jax.experimental import pallas as pl
from jax.experimental.pallas import tpu as pltpu
from jax.experimental.pallas import tpu_sc as plsc

N = 10000
HID = 128
HH = HID // 2
NT = 16
ROWS = 10240
RPT = ROWS // NT

_MESH = plsc.VectorSubcoreMesh(core_axis_name="c", subcore_axis_name="s")


def _sc_embed(emb2, hidx):
    vocab = emb2.shape[0] // 2

    @functools.partial(
        pl.kernel,
        out_type=jax.ShapeDtypeStruct((2 * ROWS, HH), jnp.float32),
        mesh=_MESH,
        compiler_params=pltpu.CompilerParams(use_tc_tiling_on_sc=False),
        scratch_types=[
            pltpu.VMEM((5, 128), jnp.int32),
            pltpu.VMEM((128, HH), jnp.float32),
        ],
    )
    def k(emb_hbm, hidx_hbm, x_hbm, idx_v, rows_v):
        cid = lax.axis_index("c")
        sid = lax.axis_index("s")
        pltpu.sync_copy(hidx_hbm.at[sid], idx_v)
        off = (cid * vocab).astype(jnp.int32)

        @pl.loop(0, 5)
        def _off(c):
            for j in range(8):
                s = pl.ds(j * 16, 16)
                idx_v[c, s] = idx_v[c, s] + off

        for c in range(5):
            pltpu.sync_copy(emb_hbm.at[idx_v.at[c]], rows_v)
            pltpu.sync_copy(
                rows_v,
                x_hbm.at[pl.ds(cid * ROWS + sid * RPT + c * 128, 128)])

    return k(emb2, hidx)


NB = 64


def _sc_agg(x2, src, dst, nchunk, page_rows):
    nblk = nchunk // NB

    @functools.partial(
        pl.kernel,
        out_type=jax.ShapeDtypeStruct((2 * ROWS, HH), jnp.float32),
        mesh=_MESH,
        compiler_params=pltpu.CompilerParams(use_tc_tiling_on_sc=False),
        scratch_types=[
            pltpu.VMEM((NB, 128), jnp.int32),
            pltpu.VMEM((NB, 128), jnp.int32),
            [pltpu.VMEM((128, HH), jnp.float32)] * 8,
            pltpu.VMEM((64, HH), jnp.float32),
            pltpu.VMEM_SHARED((ROWS, HH), jnp.float32),
            [pltpu.SemaphoreType.DMA] * 8,
            [pltpu.SemaphoreType.DMA] * 8,
        ],
    )
    def k(x_hbm, src_hbm, dst_hbm, agg_hbm, srcs_v, dsts_v, rows,
          zbuf, agg_sh, gsem, ssem):
        cid = lax.axis_index("c")
        sid = lax.axis_index("s")

        @pl.loop(0, 64)
        def _zero(r):
            for j in range(HH // 16):
                zbuf[r, pl.ds(j * 16, 16)] = jnp.zeros((16,), jnp.float32)

        @pl.loop(0, RPT // 64)
        def _init(t):
            pltpu.sync_copy(zbuf, agg_sh.at[pl.ds(sid * RPT + t * 64, 64)])

        plsc.subcore_barrier()
        off = (cid * page_rows).astype(jnp.int32)

        @pl.loop(0, nblk)
        def _blk(b):
            base = sid * nchunk + b * NB
            pltpu.sync_copy(src_hbm.at[pl.ds(base, NB)], srcs_v)
            pltpu.sync_copy(dst_hbm.at[pl.ds(base, NB)], dsts_v)

            @pl.loop(0, NB)
            def _off(c):
                for j in range(8):
                    s = pl.ds(j * 16, 16)
                    srcs_v[c, s] = srcs_v[c, s] + off

            def fire_g(c, q):
                pltpu.async_copy(x_hbm.at[srcs_v.at[c]], rows[q], gsem[q])

            def drain_g(c, q):
                pltpu.make_async_copy(x_hbm.at[srcs_v.at[c]], rows[q],
                                      gsem[q]).wait()

            def fire_s(c, q):
                pltpu.async_copy(rows[q], agg_sh.at[dsts_v.at[c]], ssem[q],
                                 add=True)

            def drain_s(c, q):
                pltpu.make_async_copy(rows[q], agg_sh.at[dsts_v.at[c]],
                                      ssem[q]).wait()

            for q0 in range(4):
                fire_g(jnp.int32(q0), q0)

            @pl.loop(0, NB // 8)
            def _edges(c8):
                for kq in range(8):
                    c = 8 * c8 + kq
                    drain_g(c, kq)
                    if kq < 4:
                        @pl.when(c >= 4)
                        def _():
                            drain_s(c - 4, kq + 4)

                        fire_g(c + 4, kq + 4)
                    else:
                        drain_s(c - 4, kq - 4)

                        @pl.when(c + 4 < NB)
                        def _():
                            fire_g(c + 4, kq - 4)

                    fire_s(c, kq)

            for q0 in range(4):
                drain_s(jnp.int32(NB - 4 + q0), 4 + q0)

        plsc.subcore_barrier()
        pltpu.sync_copy(agg_sh.at[pl.ds(sid * RPT, RPT)],
                        agg_hbm.at[pl.ds(cid * ROWS + sid * RPT, RPT)])

    return k(x2, src, dst)


def _bn_relu(hh, g, b, relu=True):
    m = jnp.mean(hh, axis=0, keepdims=True)
    v = jnp.mean((hh - m) ** 2, axis=0, keepdims=True)
    out = (hh - m) * lax.rsqrt(v + 1e-5) * g + b
    return jnp.maximum(out, 0.0) if relu else out


def _split_cols(ref, rows):
    return jnp.concatenate([ref[0:N, :], ref[rows:rows + N, :]], axis=1)


def _tc_layer1(x2, agg, w0, w1, g1, b1, g2, b2):
    def body(x_ref, agg_ref, w0_ref, w1_ref, g1_ref, b1_ref, g2_ref, b2_ref,
             out_ref):
        xv = _split_cols(x_ref, ROWS)
        rst = 2.0 * xv + _split_cols(agg_ref, ROWS)
        hh = jnp.dot(rst, w0_ref[...].T, preferred_element_type=jnp.float32)
        hh = _bn_relu(hh, g1_ref[...], b1_ref[...])
        hh = jnp.dot(hh, w1_ref[...].T, preferred_element_type=jnp.float32)
        hh = _bn_relu(hh, g2_ref[...], b2_ref[...], relu=False)
        x1 = jnp.maximum(hh, 0.0)
        out_ref[0:N, :] = x1[:, :HH]
        out_ref[N:2 * N, :] = x1[:, HH:]

    return pl.pallas_call(
        body, out_shape=jax.ShapeDtypeStruct((2 * N, HH), jnp.float32),
    )(x2, agg, w0, w1, g1, b1, g2, b2)


def _tc_layer2(x2, x12, agg, w0, w1, g1, b1, g2, b2, wu, wv, wb):
    def body(x_ref, x1_ref, agg_ref, w0_ref, w1_ref, g1_ref, b1_ref, g2_ref,
             b2_ref, wu_ref, wv_ref, wb_ref, a_ref, b_ref):
        x1v = _split_cols(x1_ref, N)
        rst = 2.0 * x1v + _split_cols(agg_ref, ROWS)
        hh = jnp.dot(rst, w0_ref[...].T, preferred_element_type=jnp.float32)
        hh = _bn_relu(hh, g1_ref[...], b1_ref[...])
        hh = jnp.dot(hh, w1_ref[...].T, preferred_element_type=jnp.float32)
        hh = _bn_relu(hh, g2_ref[...], b2_ref[...], relu=False)
        x2v = jnp.maximum(hh, 0.0)
        xsum = _split_cols(x_ref, ROWS) + x1v + x2v
        a_ref[...] = jnp.dot(xsum, wu_ref[...].T,
                             preferred_element_type=jnp.float32) + wb_ref[...]
        b_ref[...] = jnp.dot(xsum, wv_ref[...].T,
                             preferred_element_type=jnp.float32)

    return pl.pallas_call(
        body,
        out_shape=[jax.ShapeDtypeStruct((N, HID), jnp.float32),
                   jax.ShapeDtypeStruct((N, HID), jnp.float32)],
    )(x2, x12, agg, w0, w1, g1, b1, g2, b2, wu, wv, wb)


def _sc_score(a, b, u2, v2, nchunk):
    e_out = nchunk * 128
    qq, rr = divmod(nchunk, 32)
    niter = qq + (1 if rr else 0)
    npair = (niter + 2) // 2

    @functools.partial(
        pl.kernel,
        out_type=jax.ShapeDtypeStruct((e_out, HID), jnp.float32),
        mesh=_MESH,
        compiler_params=pltpu.CompilerParams(use_tc_tiling_on_sc=False),
        scratch_types=[
            [pltpu.VMEM((128,), jnp.int32)] * 2,
            [pltpu.VMEM((128,), jnp.int32)] * 2,
            [pltpu.VMEM((128, HID), jnp.float32)] * 2,
            [pltpu.VMEM((128, HID), jnp.float32)] * 2,
            [pltpu.SemaphoreType.DMA] * 2,
            [pltpu.SemaphoreType.DMA] * 2,
            [pltpu.SemaphoreType.DMA] * 2,
        ],
    )
    def k(a_hbm, b_hbm, u_hbm, v_hbm, out_hbm, uidx, vidx, bufa, bufb,
          isem, gsem, osem):
        w = lax.axis_index("s") * 2 + lax.axis_index("c")
        cnt = jnp.where(w < rr, qq + 1, qq).astype(jnp.int32)
        base = (w * qq + jnp.minimum(w, rr)).astype(jnp.int32)

        def fire_idx(i, p):
            @pl.when(i < cnt)
            def _():
                g = base + i
                pltpu.async_copy(u_hbm.at[g], uidx[p], isem[p])
                pltpu.async_copy(v_hbm.at[g], vidx[p], isem[p])

        def drain_idx(i, p):
            @pl.when(i < cnt)
            def _():
                g = base + i
                pltpu.make_async_copy(u_hbm.at[g], uidx[p], isem[p]).wait()
                pltpu.make_async_copy(v_hbm.at[g], vidx[p], isem[p]).wait()

        def fire_gath(i, p):
            @pl.when(i < cnt)
            def _():
                pltpu.async_copy(a_hbm.at[uidx[p]], bufa[p], gsem[p])
                pltpu.async_copy(b_hbm.at[vidx[p]], bufb[p], gsem[p])

        def drain_gath(i, p):
            @pl.when(i < cnt)
            def _():
                pltpu.make_async_copy(a_hbm.at[uidx[p]], bufa[p],
                                      gsem[p]).wait()
                pltpu.make_async_copy(b_hbm.at[vidx[p]], bufb[p],
                                      gsem[p]).wait()

        def fire_out(i, p):
            @pl.when(i < cnt)
            def _():
                g = base + i
                pltpu.async_copy(bufa[p], out_hbm.at[pl.ds(g * 128, 128)],
                                 osem[p])

        def drain_out(i, p):
            @pl.when((i >= 0) & (i < cnt))
            def _():
                g = base + i
                pltpu.make_async_copy(bufa[p],
                                      out_hbm.at[pl.ds(g * 128, 128)],
                                      osem[p]).wait()

        def step(i, p):
            drain_out(i - 1, 1 - p)
            drain_idx(i + 1, 1 - p)
            fire_gath(i + 1, 1 - p)
            drain_gath(i, p)
            fire_idx(i + 2, p)

            @pl.when(i < cnt)
            def _():
                @pl.loop(0, 32)
                def _add(r4):
                    for rr2 in range(4):
                        r = 4 * r4 + rr2
                        for j in range(8):
                            s = pl.ds(j * 16, 16)
                            bufa[p][r, s] = bufa[p][r, s] + bufb[p][r, s]

            fire_out(i, p)

        fire_idx(jnp.int32(0), 0)
        fire_idx(jnp.int32(1), 1)
        drain_idx(jnp.int32(0), 0)
        fire_gath(jnp.int32(0), 0)

        @pl.loop(0, npair)
        def _pair(c2):
            i = 2 * c2
            step(i, 0)
            step(i + 1, 1)

        last = jnp.int32(2 * npair - 1)
        drain_out(last, 1)

    return k(a, b, u2, v2)


def kernel(params, h, edge_index):
    u = edge_index[0]
    v = edge_index[1]
    E = u.shape[0]
    emb = params["emb"]
    vocab = emb.shape[0]

    emb2 = jnp.concatenate([emb[:, :HH], emb[:, HH:]], axis=0)
    h_pad = jnp.concatenate(
        [h, jnp.zeros((ROWS - N,), jnp.int32)]).reshape(NT, 5, 128)
    x2 = _sc_embed(emb2, h_pad)

    ee = 2 * E
    grain = NT * NB * 128
    nchunk = NB * ((ee + grain - 1) // grain)
    pad = NT * nchunk * 128 - ee
    pad_src = jnp.arange(pad, dtype=jnp.int32) % N
    pad_dst = N + (jnp.arange(pad, dtype=jnp.int32) % (ROWS - N))
    src = jnp.concatenate([u, v, pad_src]).reshape(NT * nchunk, 128)
    dst = jnp.concatenate([v, u, pad_dst]).reshape(NT * nchunk, 128)

    agg1 = _sc_agg(x2, src, dst, nchunk, ROWS)
    x12 = _tc_layer1(x2, agg1, params["w0_0"], params["w1_0"],
                     params["bng_0"].reshape(1, HID),
                     params["bnb_0"].reshape(1, HID),
                     params["obng_0"].reshape(1, HID),
                     params["obnb_0"].reshape(1, HID))
    agg2 = _sc_agg(x12, src, dst, nchunk, N)
    wu = params["W_w"][:, :HID]
    wv = params["W_w"][:, HID:]
    a_t, b_t = _tc_layer2(x2, x12, agg2, params["w0_1"], params["w1_1"],
                          params["bng_1"].reshape(1, HID),
                          params["bnb_1"].reshape(1, HID),
                          params["obng_1"].reshape(1, HID),
                          params["obnb_1"].reshape(1, HID),
                          wu, wv, params["W_b"].reshape(1, HID))

    nc_e = E // 128
    u2 = u.reshape(nc_e, 128)
    v2 = v.reshape(nc_e, 128)
    return _sc_score(a_t, b_t, u2, v2, nc_e)
